# Initial kernel scaffold; baseline (speedup 1.0000x reference)
import jax
import jax.numpy as jnp
from jax.experimental import pallas as pl


def kernel(feat, coords, knn_idx, W, b):
    raise NotImplementedError("write your pallas kernel here")


# speedup vs baseline: 2.5300x; 2.5300x over previous
import functools

import jax
import jax.numpy as jnp
from jax import lax
from jax.experimental import pallas as pl
from jax.experimental.pallas import tpu as pltpu
from jax.experimental.pallas import tpu_sc as plsc

_NC = 2
_NS = 16
_NW = _NC * _NS
_L = 16

_CH = 4
_NBUF = 2


def _sc_gather_sums(feat, cc, idx_flat, n_pad, k):
    n, c = feat.shape
    npw = n_pad // _NW
    nchunk = npw // _CH
    g = _CH * k
    nvf = c // _L

    mesh = plsc.VectorSubcoreMesh(
        core_axis_name="c", subcore_axis_name="s",
        num_cores=_NC, num_subcores=_NS)

    @functools.partial(
        pl.kernel,
        out_type=(
            jax.ShapeDtypeStruct((n_pad, c), jnp.float32),
            jax.ShapeDtypeStruct((n_pad, 16), jnp.float32),
        ),
        mesh=mesh,
        scratch_types=[
            pltpu.VMEM((npw * k,), jnp.int32),
            pltpu.VMEM((g, c), jnp.float32),
            pltpu.VMEM((g, c), jnp.float32),
            pltpu.VMEM((g, 16), jnp.float32),
            pltpu.VMEM((g, 16), jnp.float32),
            pltpu.VMEM((npw, c), jnp.float32),
            pltpu.VMEM((npw, 16), jnp.float32),
            pltpu.SemaphoreType.DMA,
            pltpu.SemaphoreType.DMA,
            pltpu.SemaphoreType.DMA,
            pltpu.SemaphoreType.DMA,
        ],
        compiler_params=pltpu.CompilerParams(use_tc_tiling_on_sc=False),
    )
    def sc_kernel(feat_h, cc_h, idx_h, outf_h, outc_h,
                  idx_v, rf0, rf1, rc0, rc1, sumf_v, sumc_v,
                  sf0, sf1, sc0, sc1):
        rfs, rcs = [rf0, rf1], [rc0, rc1]
        sfs, scs = [sf0, sf1], [sc0, sc1]
        wid = lax.axis_index("s") * _NC + lax.axis_index("c")
        nbase = wid * npw

        pltpu.sync_copy(idx_h.at[pl.ds(nbase * k, npw * k)], idx_v)

        def start(chunk, b):
            idx_sl = idx_v.at[pl.ds(chunk * g, g)]
            pltpu.async_copy(feat_h.at[idx_sl], rfs[b], sfs[b])
            pltpu.async_copy(cc_h.at[idx_sl], rcs[b], scs[b])

        def wait(chunk, b):
            idx_sl = idx_v.at[pl.ds(chunk * g, g)]
            pltpu.make_async_copy(feat_h.at[idx_sl], rfs[b], sfs[b]).wait()
            pltpu.make_async_copy(cc_h.at[idx_sl], rcs[b], scs[b]).wait()

        for b in range(_NBUF):
            start(b, b)

        @pl.loop(0, nchunk, step=_NBUF)
        def _chunks(c0):
            for b in range(_NBUF):
                ci = c0 + b
                wait(ci, b)
                for j in range(_CH):
                    r0 = j * k
                    accf = [rfs[b][r0, pl.ds(v * _L, _L)] for v in range(nvf)]
                    accc = rcs[b][r0, :]
                    for kk in range(1, k):
                        for v in range(nvf):
                            accf[v] = accf[v] + rfs[b][r0 + kk, pl.ds(v * _L, _L)]
                        accc = accc + rcs[b][r0 + kk, :]
                    node = ci * _CH + j
                    for v in range(nvf):
                        sumf_v[node, pl.ds(v * _L, _L)] = accf[v]
                    sumc_v[node, :] = accc
                nxt = ci + _NBUF

                @pl.when(nxt < nchunk)
                def _():
                    start(nxt, b)

        pltpu.sync_copy(sumf_v, outf_h.at[pl.ds(nbase, npw)])
        pltpu.sync_copy(sumc_v, outc_h.at[pl.ds(nbase, npw)])

    return sc_kernel(feat, cc, idx_flat)


def _tc_dense(feat, sums_f, sums_c, cc, w1, w2k, w3m, w3s, b2, inv_k, br):
    n, c = feat.shape

    def body(f_ref, sf_ref, sc_ref, cc_ref, w1_ref, w2_ref, w3m_ref,
             w3s_ref, b_ref, o_ref):
        f = f_ref[...]
        q = sc_ref[...] * inv_k
        q1 = q[:, :8]
        q2 = q[:, 8:]
        ci = cc_ref[...][:, :8]
        rel_m = q1 - ci
        var = jnp.maximum(q2 - q1 * q1, 0.0)
        rel_s = jnp.sqrt(var)
        acc = jnp.dot(f, w1_ref[...], preferred_element_type=jnp.float32)
        acc += jnp.dot(sf_ref[...], w2_ref[...],
                       preferred_element_type=jnp.float32)
        acc += jnp.dot(rel_m, w3m_ref[...], preferred_element_type=jnp.float32)
        acc += jnp.dot(rel_s, w3s_ref[...], preferred_element_type=jnp.float32)
        acc += b_ref[...]
        o_ref[...] = jnp.maximum(acc, 0.0)

    nb = n // br
    row = lambda i: (i, 0)
    fixed = lambda i: (0, 0)
    return pl.pallas_call(
        body,
        grid=(nb,),
        in_specs=[
            pl.BlockSpec((br, c), row),
            pl.BlockSpec((br, c), row),
            pl.BlockSpec((br, 16), row),
            pl.BlockSpec((br, 16), row),
            pl.BlockSpec((c, c), fixed),
            pl.BlockSpec((c, c), fixed),
            pl.BlockSpec((8, c), fixed),
            pl.BlockSpec((8, c), fixed),
            pl.BlockSpec((1, c), fixed),
        ],
        out_specs=pl.BlockSpec((br, c), row),
        out_shape=jax.ShapeDtypeStruct((n, c), jnp.float32),
    )(feat, sums_f, sums_c, cc, w1, w2k, w3m, w3s, b2)


def kernel(feat, coords, knn_idx, W, b):
    n, c = feat.shape
    k = knn_idx.shape[1]
    inv_k = 1.0 / k

    chunk_nodes = _NW * _CH * _NBUF
    n_pad = ((n + chunk_nodes - 1) // chunk_nodes) * chunk_nodes

    c8 = jnp.pad(coords, ((0, 0), (0, 8 - coords.shape[1])))
    cc = jnp.concatenate([c8, c8 * c8], axis=1)

    idx_flat = jnp.pad(knn_idx.reshape(-1), (0, (n_pad - n) * k))

    sums_f, sums_c = _sc_gather_sums(feat, cc, idx_flat, n_pad, k)

    w1 = W[:c]
    w2k = W[c:2 * c] * inv_k
    w3m = jnp.zeros((8, c), jnp.float32).at[:3].set(W[2 * c:2 * c + 3])
    w3s = jnp.zeros((8, c), jnp.float32).at[:3].set(W[2 * c + 3:2 * c + 6])
    b2 = b.reshape(1, c)

    br = 1000 if n % 1000 == 0 else 8
    return _tc_dense(feat, sums_f[:n], sums_c[:n], cc, w1, w2k, w3m, w3s,
                     b2, inv_k, br)
